# Initial kernel scaffold; baseline (speedup 1.0000x reference)
import functools

import jax
import jax.numpy as jnp
from jax import lax
from jax.experimental import pallas as pl
from jax.experimental.pallas import tpu as pltpu
from jax.experimental.pallas import tpu_sc as plsc

_MAX_REL = 20
_N = 1024
_D = 64
_V = 2 * _MAX_REL + 1
_STRIP = 2 * _N - 1
_FILL = _N - _MAX_REL - 1

_NC, _NS = 2, 16
_NW = _NC * _NS
_ROWS_PER_W = _N // _NW
_INFLIGHT = 8


def _doubling_fill(strip, base, total):
    have = 1
    while have < total:
        m = min(have, total - have)
        pltpu.sync_copy(
            strip.at[pl.ds(base, m), :],
            strip.at[pl.ds(base + have, m), :],
        )
        have += m


@functools.partial(
    pl.kernel,
    out_type=jax.ShapeDtypeStruct((_N, _N, _D), jnp.float32),
    mesh=plsc.VectorSubcoreMesh(core_axis_name="c", subcore_axis_name="s"),
    scratch_types=[
        pltpu.VMEM((_STRIP, _D), jnp.float32),
        pltpu.SemaphoreType.DMA,
    ],
)
def _rel_pos_sc(table_hbm, out_hbm, strip, sem):
    wid = lax.axis_index("s") * _NC + lax.axis_index("c")
    i0 = wid * _ROWS_PER_W

    pltpu.sync_copy(table_hbm.at[pl.ds(0, 1)], strip.at[pl.ds(0, 1), :])
    _doubling_fill(strip, 0, _FILL)
    pltpu.sync_copy(table_hbm, strip.at[pl.ds(_FILL, _V), :])
    pltpu.sync_copy(
        table_hbm.at[pl.ds(_V - 1, 1)],
        strip.at[pl.ds(_FILL + _V, 1), :],
    )
    _doubling_fill(strip, _FILL + _V, _FILL)

    copies = []
    for k in range(_ROWS_PER_W):
        i = i0 + k
        cp = pltpu.async_copy(
            strip.at[pl.ds(_N - 1 - i, _N), :],
            out_hbm.at[i],
            sem,
        )
        copies.append(cp)
        if k >= _INFLIGHT:
            copies[k - _INFLIGHT].wait()
    for cp in copies[_ROWS_PER_W - _INFLIGHT:]:
        cp.wait()


@jax.jit
def _run(table):
    return _rel_pos_sc(table)


def kernel(seq_len, table):
    del seq_len
    return _run(table)


# speedup vs baseline: 5.5187x; 5.5187x over previous
import functools

import jax
import jax.numpy as jnp
from jax import lax
from jax.experimental import pallas as pl
from jax.experimental.pallas import tpu as pltpu
from jax.experimental.pallas import tpu_sc as plsc

_MAX_REL = 20
_N = 1024
_D = 64
_V = 2 * _MAX_REL + 1
_STRIP = 2 * _N - 1
_FILL = _N - _MAX_REL - 1
_LANES = 16

_NC, _NS = 2, 16
_NW = _NC * _NS
_ROWS_PER_W = _N // _NW
_INFLIGHT = 8


@functools.partial(
    pl.kernel,
    out_type=jax.ShapeDtypeStruct((_N, _N, _D), jnp.float32),
    mesh=plsc.VectorSubcoreMesh(core_axis_name="c", subcore_axis_name="s"),
    scratch_types=[
        pltpu.VMEM((_STRIP, _D), jnp.float32),
        pltpu.SemaphoreType.DMA,
    ],
    compiler_params=pltpu.CompilerParams(use_tc_tiling_on_sc=False),
)
def _rel_pos_sc(table_hbm, out_hbm, strip, sem):
    wid = lax.axis_index("s") * _NC + lax.axis_index("c")
    i0 = wid * _ROWS_PER_W

    pltpu.sync_copy(table_hbm, strip.at[pl.ds(_FILL, _V), :])
    row0 = [
        strip[_FILL, pl.ds(c * _LANES, _LANES)] for c in range(_D // _LANES)
    ]
    row40 = [
        strip[_FILL + _V - 1, pl.ds(c * _LANES, _LANES)]
        for c in range(_D // _LANES)
    ]

    def _fill(r, carry):
        for c in range(_D // _LANES):
            strip[r, pl.ds(c * _LANES, _LANES)] = row0[c]
            strip[_FILL + _V + r, pl.ds(c * _LANES, _LANES)] = row40[c]
        return carry

    lax.fori_loop(0, _FILL, _fill, 0)

    copies = []
    for k in range(_ROWS_PER_W):
        i = i0 + k
        cp = pltpu.async_copy(
            strip.at[pl.ds(_N - 1 - i, _N), :],
            out_hbm.at[i],
            sem,
        )
        copies.append(cp)
        if k >= _INFLIGHT:
            copies[k - _INFLIGHT].wait()
    for cp in copies[_ROWS_PER_W - _INFLIGHT:]:
        cp.wait()


@jax.jit
def _run(table):
    return _rel_pos_sc(table)


def kernel(seq_len, table):
    del seq_len
    return _run(table)


# speedup vs baseline: 36.1198x; 6.5450x over previous
import functools

import jax
import jax.numpy as jnp
from jax import lax
from jax.experimental import pallas as pl
from jax.experimental.pallas import tpu as pltpu
from jax.experimental.pallas import tpu_sc as plsc

_MAX_REL = 20
_N = 1024
_D = 64
_V = 2 * _MAX_REL + 1
_LANES = 16

_NC, _NS = 2, 16
_NW = _NC * _NS
_ROWS_PER_W = _N // _NW

_TCOLS = 1280
_BCOLS = 48
_DT = _D // 8
_JT = _N // 128


@functools.partial(
    pl.kernel,
    out_type=jax.ShapeDtypeStruct((_N, _DT, _JT, 8, 128), jnp.float32),
    mesh=plsc.VectorSubcoreMesh(core_axis_name="c", subcore_axis_name="s"),
    scratch_types=[
        pltpu.VMEM((_D, _TCOLS), jnp.float32),
        pltpu.VMEM((_D, _BCOLS), jnp.float32),
        pltpu.SemaphoreType.DMA,
    ],
    compiler_params=pltpu.CompilerParams(
        use_tc_tiling_on_sc=False, needs_layout_passes=False
    ),
)
def _rel_pos_sc(bands_hbm, out_hbm, stript, tband, sem):
    wid = lax.axis_index("s") * _NC + lax.axis_index("c")
    a = lax.rem(wid, 8)
    b = lax.div(wid, 8)
    t_b = 228 + 256 * b + a
    s = lax.rem(a + 4, 8)
    t_w = t_b - s

    pltpu.sync_copy(bands_hbm.at[s], tband)

    iotas = [
        lax.iota(jnp.int32, _LANES) + c * _LANES
        for c in range(_TCOLS // _LANES)
    ]
    thresh = t_b + _MAX_REL

    def _build_row(d, carry):
        dsplat = jnp.full((_LANES,), d, dtype=jnp.int32)
        splat0 = plsc.load_gather(
            tband, [dsplat, jnp.full((_LANES,), s, dtype=jnp.int32)]
        )
        splat40 = plsc.load_gather(
            tband, [dsplat, jnp.full((_LANES,), s + _V - 1, dtype=jnp.int32)]
        )
        for c in range(_TCOLS // _LANES):
            stript[d, pl.ds(c * _LANES, _LANES)] = jnp.where(
                iotas[c] < thresh, splat0, splat40
            )
        for c in range(3):
            stript[d, pl.ds(t_w + c * _LANES, _LANES)] = tband[
                d, pl.ds(c * _LANES, _LANES)
            ]
        return carry

    lax.fori_loop(0, _D, _build_row, 0)

    def _emit_row(m, carry):
        i = a + 256 * b + 8 * m
        t0 = 248 - 8 * m
        copies = []
        for dt in range(_DT):
            for jt in range(_JT):
                copies.append(
                    pltpu.async_copy(
                        stript.at[
                            pl.ds(dt * 8, 8), pl.ds(t0 + jt * 128, 128)
                        ],
                        out_hbm.at[i, dt, jt],
                        sem,
                    )
                )
        for cp in copies:
            cp.wait()
        return carry

    lax.fori_loop(0, _ROWS_PER_W, _emit_row, 0)


@jax.jit
def _run(table):
    tablet = table.T
    col0 = tablet[:, :1]
    col40 = tablet[:, _V - 1 :]
    bands = jnp.stack(
        [
            jnp.concatenate(
                [
                    jnp.broadcast_to(col0, (_D, sh)),
                    tablet,
                    jnp.broadcast_to(col40, (_D, _BCOLS - _V - sh)),
                ],
                axis=1,
            )
            for sh in range(8)
        ]
    )
    phys = _rel_pos_sc(bands)
    return phys.transpose(0, 2, 4, 1, 3).reshape(_N, _N, _D)


def kernel(seq_len, table):
    del seq_len
    return _run(table)
